# baseline (speedup 1.0000x reference)
import functools

import jax
import jax.numpy as jnp
from jax import lax
from jax.experimental import pallas as pl
from jax.experimental.pallas import tpu as pltpu
from jax.experimental.pallas import tpu_sc as plsc

NENTITY = 1000000
NREL = 1000
D = 64
B = 16384

NC = 2
NS = 16
L = 16
NW = NC * NS
BPW = B // NW
CHUNK = 64
NCHUNK = BPW // CHUNK


def _weights_body(relx_ref, rely_ref, alpha_ref, w1e_ref, w1o_ref, w2e_ref, w2o_ref):
    x = jnp.sign(relx_ref[...])
    y = jnp.sign(rely_ref[...])
    a = jnp.sign(alpha_ref[...])
    u = (x + y) * 0.5
    v = (x - y) * 0.5
    w1e_ref[...] = u
    w1o_ref[...] = u * a
    w2e_ref[...] = -(v * a)
    w2o_ref[...] = v


_weights_tc = pl.pallas_call(
    _weights_body,
    out_shape=[jax.ShapeDtypeStruct((NREL, D), jnp.float32)] * 4,
)


def _sc_body(ent, w1, w2, hidx, tidx, ridx, out,
             hidx_v, tidx_v, ridx_v, hbuf, tbuf, w1buf, w2buf, accbuf, sbuf, sem):
    wid = lax.axis_index("s") * NC + lax.axis_index("c")
    base = wid * BPW
    pltpu.sync_copy(hidx.at[pl.ds(base, BPW)], hidx_v)
    pltpu.sync_copy(tidx.at[pl.ds(base, BPW)], tidx_v)
    pltpu.sync_copy(ridx.at[pl.ds(base, BPW)], ridx_v)

    lane = lax.iota(jnp.int32, L)
    evens = [2 * lane + (32 * m) for m in range(D // L)]

    for ci in range(NCHUNK):
        cbase = ci * CHUNK
        cps = [
            pltpu.async_copy(ent.at[hidx_v.at[pl.ds(cbase, CHUNK)]], hbuf, sem),
            pltpu.async_copy(ent.at[tidx_v.at[pl.ds(cbase, CHUNK)]], tbuf, sem),
            pltpu.async_copy(w1.at[ridx_v.at[pl.ds(cbase, CHUNK)]], w1buf, sem),
            pltpu.async_copy(w2.at[ridx_v.at[pl.ds(cbase, CHUNK)]], w2buf, sem),
        ]
        for cp in cps:
            cp.wait()

        def row_step(r, _):
            hr, tr = hbuf.at[r], tbuf.at[r]
            w1r, w2r = w1buf.at[r], w2buf.at[r]
            acc = jnp.zeros((L,), jnp.float32)
            for m in range(D // L):
                e = evens[m]
                o = e + 1
                h0 = plsc.load_gather(hr, [e])
                h1 = plsc.load_gather(hr, [o])
                t0 = plsc.load_gather(tr, [e])
                t1 = plsc.load_gather(tr, [o])
                w1e = plsc.load_gather(w1r, [e])
                w1o = plsc.load_gather(w1r, [o])
                w2e = plsc.load_gather(w2r, [e])
                w2o = plsc.load_gather(w2r, [o])
                s = (h0 * t0 * w1e + h1 * t1 * w1o
                     + h0 * t1 * w2e + h1 * t0 * w2o)
                acc = acc + s * s
            accbuf[pl.ds(pl.multiple_of(r * L, L), L)] = acc
            return _

        lax.fori_loop(0, CHUNK, row_step, 0)

        for g in range(CHUNK // L):
            rb = (g * L + lane) * L
            x = plsc.load_gather(accbuf, [rb])
            for k in range(1, L):
                x = x + plsc.load_gather(accbuf, [rb + k])
            yi = jnp.int32(0x5F3759DF) - (plsc.bitcast(x, jnp.int32) >> 1)
            y = plsc.bitcast(yi, jnp.float32)
            for _ in range(3):
                y = y * (1.5 - 0.5 * x * y * y)
            sbuf[pl.ds(cbase + g * L, L)] = jnp.where(x > 0.0, x * y, 0.0)

    pltpu.sync_copy(sbuf, out.at[pl.ds(base, BPW)])


_sc_score = functools.partial(
    pl.kernel,
    out_type=jax.ShapeDtypeStruct((B,), jnp.float32),
    mesh=plsc.VectorSubcoreMesh(core_axis_name="c", subcore_axis_name="s"),
    compiler_params=pltpu.CompilerParams(needs_layout_passes=False),
    scratch_types=[
        pltpu.VMEM((BPW,), jnp.int32),
        pltpu.VMEM((BPW,), jnp.int32),
        pltpu.VMEM((BPW,), jnp.int32),
        pltpu.VMEM((CHUNK, 2 * D), jnp.float32),
        pltpu.VMEM((CHUNK, 2 * D), jnp.float32),
        pltpu.VMEM((CHUNK, 2 * D), jnp.float32),
        pltpu.VMEM((CHUNK, 2 * D), jnp.float32),
        pltpu.VMEM((CHUNK * L,), jnp.float32),
        pltpu.VMEM((BPW,), jnp.float32),
        pltpu.SemaphoreType.DMA,
    ],
)(_sc_body)


def kernel(head_idx, relation_idx, tail_idx, entity_embedding,
           relation_embedding, alpha_embedding):
    ent = entity_embedding.reshape(NENTITY, 2 * D)
    relx = relation_embedding[:, :, 0]
    rely = relation_embedding[:, :, 1]
    w1e, w1o, w2e, w2o = _weights_tc(relx, rely, alpha_embedding)
    w1 = jnp.stack([w1e, w1o], axis=-1).reshape(NREL, 2 * D)
    w2 = jnp.stack([w2e, w2o], axis=-1).reshape(NREL, 2 * D)
    return _sc_score(ent, w1, w2,
                     head_idx.astype(jnp.int32),
                     tail_idx.astype(jnp.int32),
                     relation_idx.astype(jnp.int32))


# speedup vs baseline: 2.5331x; 2.5331x over previous
import functools

import jax
import jax.numpy as jnp
from jax import lax
from jax.experimental import pallas as pl
from jax.experimental.pallas import tpu as pltpu
from jax.experimental.pallas import tpu_sc as plsc

NENTITY = 1000000
NREL = 1000
D = 64
B = 16384

NC = 2
NS = 16
L = 16
NW = NC * NS
BPW = B // NW
CHUNK = 64
NCHUNK = BPW // CHUNK


def _weights_body(relx_ref, rely_ref, alpha_ref, w1e_ref, w1o_ref, w2e_ref, w2o_ref):
    x = jnp.sign(relx_ref[...])
    y = jnp.sign(rely_ref[...])
    a = jnp.sign(alpha_ref[...])
    u = (x + y) * 0.5
    v = (x - y) * 0.5
    w1e_ref[...] = u
    w1o_ref[...] = u * a
    w2e_ref[...] = -(v * a)
    w2o_ref[...] = v


_weights_tc = pl.pallas_call(
    _weights_body,
    out_shape=[jax.ShapeDtypeStruct((NREL, D), jnp.float32)] * 4,
)


def _sc_body(ent, w, hidx, tidx, ridx, out,
             hidx_v, tidx_v, ridx_v, hbuf, tbuf, wbuf, accbuf, sbuf, sem, wsem):
    wid = lax.axis_index("s") * NC + lax.axis_index("c")
    base = wid * BPW
    pltpu.sync_copy(hidx.at[pl.ds(base, BPW)], hidx_v)
    pltpu.sync_copy(tidx.at[pl.ds(base, BPW)], tidx_v)
    pltpu.sync_copy(ridx.at[pl.ds(base, BPW)], ridx_v)

    lane = lax.iota(jnp.int32, L)

    for ci in range(NCHUNK):
        cbase = ci * CHUNK
        wcp = pltpu.async_copy(w.at[ridx_v.at[pl.ds(cbase, CHUNK)]], wbuf, wsem)

        def dma_rows(g, _):
            hv = hidx_v[pl.ds(cbase + g * L, L)]
            tv = tidx_v[pl.ds(cbase + g * L, L)]
            for l in range(L):
                pltpu.async_copy(ent.at[hv[l]], hbuf.at[g * L + l], sem)
                pltpu.async_copy(ent.at[tv[l]], tbuf.at[g * L + l], sem)
            return _

        lax.fori_loop(0, CHUNK // L, dma_rows, 0)
        pltpu.make_async_copy(ent.at[pl.ds(0, CHUNK)], hbuf, sem).wait()
        pltpu.make_async_copy(ent.at[pl.ds(0, CHUNK)], tbuf, sem).wait()
        wcp.wait()

        def row_step(r, _):
            acc = jnp.zeros((L,), jnp.float32)
            for j in range(D // L):
                c = pl.ds(j * L, L)
                cw2 = pl.ds(D + j * L, L)
                h0 = hbuf[r, 0, c]
                h1 = hbuf[r, 1, c]
                t0 = tbuf[r, 0, c]
                t1 = tbuf[r, 1, c]
                w1e = wbuf[r, 0, c]
                w1o = wbuf[r, 1, c]
                w2e = wbuf[r, 0, cw2]
                w2o = wbuf[r, 1, cw2]
                s = (h0 * t0 * w1e + h1 * t1 * w1o
                     + h0 * t1 * w2e + h1 * t0 * w2o)
                acc = acc + s * s
            accbuf[pl.ds(pl.multiple_of(r * L, L), L)] = acc
            return _

        lax.fori_loop(0, CHUNK, row_step, 0)

        for g in range(CHUNK // L):
            rb = (g * L + lane) * L
            x = plsc.load_gather(accbuf, [rb])
            for k in range(1, L):
                x = x + plsc.load_gather(accbuf, [rb + k])
            yi = jnp.int32(0x5F3759DF) - (plsc.bitcast(x, jnp.int32) >> 1)
            y = plsc.bitcast(yi, jnp.float32)
            for _ in range(3):
                y = y * (1.5 - 0.5 * x * y * y)
            sbuf[pl.ds(cbase + g * L, L)] = jnp.where(x > 0.0, x * y, 0.0)

    pltpu.sync_copy(sbuf, out.at[pl.ds(base, BPW)])


_sc_score = functools.partial(
    pl.kernel,
    out_type=jax.ShapeDtypeStruct((B,), jnp.float32),
    mesh=plsc.VectorSubcoreMesh(core_axis_name="c", subcore_axis_name="s"),
    compiler_params=pltpu.CompilerParams(needs_layout_passes=False),
    scratch_types=[
        pltpu.VMEM((BPW,), jnp.int32),
        pltpu.VMEM((BPW,), jnp.int32),
        pltpu.VMEM((BPW,), jnp.int32),
        pltpu.VMEM((CHUNK, 2, D), jnp.float32),
        pltpu.VMEM((CHUNK, 2, D), jnp.float32),
        pltpu.VMEM((CHUNK, 2, 2 * D), jnp.float32),
        pltpu.VMEM((CHUNK * L,), jnp.float32),
        pltpu.VMEM((BPW,), jnp.float32),
        pltpu.SemaphoreType.DMA,
        pltpu.SemaphoreType.DMA,
    ],
)(_sc_body)


def kernel(head_idx, relation_idx, tail_idx, entity_embedding,
           relation_embedding, alpha_embedding):
    ent = entity_embedding[:, :, 0, :].transpose(0, 2, 1)
    relx = relation_embedding[:, :, 0]
    rely = relation_embedding[:, :, 1]
    w1e, w1o, w2e, w2o = _weights_tc(relx, rely, alpha_embedding)
    w = jnp.stack([jnp.concatenate([w1e, w2e], axis=1),
                   jnp.concatenate([w1o, w2o], axis=1)], axis=1)
    return _sc_score(ent, w,
                     head_idx.astype(jnp.int32),
                     tail_idx.astype(jnp.int32),
                     relation_idx.astype(jnp.int32))


# speedup vs baseline: 2.6013x; 1.0269x over previous
import functools

import jax
import jax.numpy as jnp
from jax import lax
from jax.experimental import pallas as pl
from jax.experimental.pallas import tpu as pltpu
from jax.experimental.pallas import tpu_sc as plsc

NENTITY = 1000000
NREL = 1000
D = 64
B = 16384

NC = 2
NS = 16
L = 16
NW = NC * NS
BPW = B // NW
CHUNK = 64
NCHUNK = BPW // CHUNK


def _weights_body(relx_ref, rely_ref, alpha_ref, w1e_ref, w1o_ref, w2e_ref, w2o_ref):
    x = jnp.sign(relx_ref[...])
    y = jnp.sign(rely_ref[...])
    a = jnp.sign(alpha_ref[...])
    u = (x + y) * 0.5
    v = (x - y) * 0.5
    w1e_ref[...] = u
    w1o_ref[...] = u * a
    w2e_ref[...] = -(v * a)
    w2o_ref[...] = v


_weights_tc = pl.pallas_call(
    _weights_body,
    out_shape=[jax.ShapeDtypeStruct((NREL, D), jnp.float32)] * 4,
)


def _sc_body(ent, w, hidx, tidx, ridx, out,
             hidx_v, tidx_v, ridx_v, hbuf0, tbuf0, wbuf0, hbuf1, tbuf1, wbuf1,
             accbuf, sbuf, sem0, wsem0, sem1, wsem1):
    wid = lax.axis_index("s") * NC + lax.axis_index("c")
    base = wid * BPW
    pltpu.sync_copy(hidx.at[pl.ds(base, BPW)], hidx_v)
    pltpu.sync_copy(tidx.at[pl.ds(base, BPW)], tidx_v)
    pltpu.sync_copy(ridx.at[pl.ds(base, BPW)], ridx_v)

    lane = lax.iota(jnp.int32, L)
    hbufs, tbufs, wbufs = (hbuf0, hbuf1), (tbuf0, tbuf1), (wbuf0, wbuf1)
    sems, wsems = (sem0, sem1), (wsem0, wsem1)

    def fire(ci):
        p = ci % 2
        cbase = ci * CHUNK
        pltpu.async_copy(w.at[ridx_v.at[pl.ds(cbase, CHUNK)]], wbufs[p],
                         wsems[p])

        def dma_rows(g, _):
            hv = hidx_v[pl.ds(cbase + g * L, L)]
            tv = tidx_v[pl.ds(cbase + g * L, L)]
            for l in range(L):
                pltpu.async_copy(ent.at[hv[l]], hbufs[p].at[g * L + l], sems[p])
                pltpu.async_copy(ent.at[tv[l]], tbufs[p].at[g * L + l], sems[p])
            return _

        lax.fori_loop(0, CHUNK // L, dma_rows, 0)

    fire(0)
    for ci in range(NCHUNK):
        p = ci % 2
        cbase = ci * CHUNK
        hbuf, tbuf, wbuf = hbufs[p], tbufs[p], wbufs[p]
        pltpu.make_async_copy(ent.at[pl.ds(0, CHUNK)], hbuf, sems[p]).wait()
        pltpu.make_async_copy(ent.at[pl.ds(0, CHUNK)], tbuf, sems[p]).wait()
        pltpu.make_async_copy(w.at[pl.ds(0, CHUNK)], wbuf, wsems[p]).wait()
        if ci + 1 < NCHUNK:
            fire(ci + 1)

        def row_step(r, _):
            acc = jnp.zeros((L,), jnp.float32)
            for j in range(D // L):
                c = pl.ds(j * L, L)
                cw2 = pl.ds(D + j * L, L)
                h0 = hbuf[r, 0, c]
                h1 = hbuf[r, 1, c]
                t0 = tbuf[r, 0, c]
                t1 = tbuf[r, 1, c]
                w1e = wbuf[r, 0, c]
                w1o = wbuf[r, 1, c]
                w2e = wbuf[r, 0, cw2]
                w2o = wbuf[r, 1, cw2]
                s = (h0 * t0 * w1e + h1 * t1 * w1o
                     + h0 * t1 * w2e + h1 * t0 * w2o)
                acc = acc + s * s
            accbuf[pl.ds(pl.multiple_of(r * L, L), L)] = acc
            return _

        lax.fori_loop(0, CHUNK, row_step, 0)

        for g in range(CHUNK // L):
            rb = (g * L + lane) * L
            x = plsc.load_gather(accbuf, [rb])
            for k in range(1, L):
                x = x + plsc.load_gather(accbuf, [rb + k])
            yi = jnp.int32(0x5F3759DF) - (plsc.bitcast(x, jnp.int32) >> 1)
            y = plsc.bitcast(yi, jnp.float32)
            for _ in range(3):
                y = y * (1.5 - 0.5 * x * y * y)
            sbuf[pl.ds(cbase + g * L, L)] = jnp.where(x > 0.0, x * y, 0.0)

    pltpu.sync_copy(sbuf, out.at[pl.ds(base, BPW)])


_sc_score = functools.partial(
    pl.kernel,
    out_type=jax.ShapeDtypeStruct((B,), jnp.float32),
    mesh=plsc.VectorSubcoreMesh(core_axis_name="c", subcore_axis_name="s"),
    compiler_params=pltpu.CompilerParams(needs_layout_passes=False),
    scratch_types=[
        pltpu.VMEM((BPW,), jnp.int32),
        pltpu.VMEM((BPW,), jnp.int32),
        pltpu.VMEM((BPW,), jnp.int32),
        pltpu.VMEM((CHUNK, 2, D), jnp.float32),
        pltpu.VMEM((CHUNK, 2, D), jnp.float32),
        pltpu.VMEM((CHUNK, 2, 2 * D), jnp.float32),
        pltpu.VMEM((CHUNK, 2, D), jnp.float32),
        pltpu.VMEM((CHUNK, 2, D), jnp.float32),
        pltpu.VMEM((CHUNK, 2, 2 * D), jnp.float32),
        pltpu.VMEM((CHUNK * L,), jnp.float32),
        pltpu.VMEM((BPW,), jnp.float32),
        pltpu.SemaphoreType.DMA,
        pltpu.SemaphoreType.DMA,
        pltpu.SemaphoreType.DMA,
        pltpu.SemaphoreType.DMA,
    ],
)(_sc_body)


def kernel(head_idx, relation_idx, tail_idx, entity_embedding,
           relation_embedding, alpha_embedding):
    ent = entity_embedding[:, :, 0, :].transpose(0, 2, 1)
    relx = relation_embedding[:, :, 0]
    rely = relation_embedding[:, :, 1]
    w1e, w1o, w2e, w2o = _weights_tc(relx, rely, alpha_embedding)
    w = jnp.stack([jnp.concatenate([w1e, w2e], axis=1),
                   jnp.concatenate([w1o, w2o], axis=1)], axis=1)
    return _sc_score(ent, w,
                     head_idx.astype(jnp.int32),
                     tail_idx.astype(jnp.int32),
                     relation_idx.astype(jnp.int32))
